# baseline (device time: 51274 ns/iter reference)
import jax
import jax.numpy as jnp
from jax import lax
from jax.experimental import pallas as pl
from jax.experimental.pallas import tpu as pltpu

N_Y = 4


def kernel(x):
    m_per, n = x.shape
    qrows = m_per // 4
    srows = qrows // 2

    def body(x_ref, out_ref, ys_s, y_r, xb_s, xb_r, zc_s, zc_r,
             xr_s, xr_r, zr_s, zr_r):
        my_x = lax.axis_index("x")
        my_y = lax.axis_index("y")
        my_z = lax.axis_index("z")
        zp = my_z % 2
        partner = (1 - my_x, my_y, my_z)
        buddy = (my_x, my_y, my_z - 2 * zp + 1)

        q_me = 2 * my_x + zp
        q_partner = 2 * (1 - my_x) + zp
        q_buddy = 2 * my_x + (1 - zp)
        q_diag = 2 * (1 - my_x) + (1 - zp)

        def sub(c, q, k):
            return out_ref.at[pl.ds(c * m_per + q * qrows + k * srows, srows), :]

        def copy(src, dst, ssem, rsem, dev):
            return pltpu.make_async_remote_copy(
                src_ref=src, dst_ref=dst, send_sem=ssem, recv_sem=rsem,
                device_id=dev, device_id_type=pl.DeviceIdType.MESH,
            )

        def slot(src_y):
            return jnp.where(src_y < my_y, src_y, src_y - 1)

        def sel(table):
            v = jnp.int32(table[3])
            for yy in (2, 1, 0):
                v = jnp.where(my_y == yy, table[yy], v)
            return v

        srcs = [sel(t) for t in ([1, 0, 1, 2], [2, 2, 3, 1], [3, 3, 0, 0])]

        barrier_sem = pltpu.get_barrier_semaphore()
        peers = [(my_x, (my_y + 1 + k) % N_Y, my_z) for k in range(3)]
        peers += [partner, buddy]
        for dev in peers:
            pl.semaphore_signal(
                barrier_sem, inc=1, device_id=dev,
                device_id_type=pl.DeviceIdType.MESH,
            )
        pl.semaphore_wait(barrier_sem, len(peers))

        out_ref[pl.ds(my_y * m_per + q_me * qrows, qrows), :] = (
            x_ref[pl.ds(q_me * qrows, qrows), :].astype(jnp.bfloat16)
        )

        for k in range(2):
            for t in range(3):
                y_t = (my_y + 1 + t) % N_Y
                r_slot = jnp.where(my_y < y_t, my_y, my_y - 1)
                copy(sub(my_y, q_me, k), sub(my_y, q_me, k),
                     ys_s.at[t, k], y_r.at[r_slot, k],
                     (my_x, y_t, my_z)).start()

        for d in range(1, 4):
            q = (q_me + d) % 4
            out_ref[pl.ds(my_y * m_per + q * qrows, qrows), :] = (
                x_ref[pl.ds(q * qrows, qrows), :].astype(jnp.bfloat16)
            )

        for j in range(3):
            src = srcs[j]
            s = slot(src)
            for k in range(2):
                copy(sub(src, q_me, k), sub(src, q_me, k),
                     ys_s.at[0, k], y_r.at[s, k],
                     (my_x, src, my_z)).wait_recv()
                copy(sub(src, q_me, k), sub(src, q_me, k),
                     xb_s.at[s, k], xb_r.at[s, k], partner).start()
                copy(sub(src, q_me, k), sub(src, q_me, k),
                     zc_s.at[s, k], zc_r.at[s, k], buddy).start()

        for j in range(3):
            src = srcs[j]
            s = slot(src)
            copy(sub(src, q_partner, 0), sub(src, q_partner, 0),
                 xb_s.at[s, 0], xb_r.at[s, 0], partner).wait_recv()
            copy(sub(src, q_partner, 0), sub(src, q_partner, 0),
                 zr_s.at[s], zr_r.at[s], buddy).start()
            copy(sub(src, q_buddy, 1), sub(src, q_buddy, 1),
                 zc_s.at[s, 1], zc_r.at[s, 1], buddy).wait_recv()
            copy(sub(src, q_buddy, 1), sub(src, q_buddy, 1),
                 xr_s.at[s], xr_r.at[s], partner).start()
            copy(sub(src, q_partner, 1), sub(src, q_partner, 1),
                 xb_s.at[s, 1], xb_r.at[s, 1], partner).wait_recv()
            copy(sub(src, q_buddy, 0), sub(src, q_buddy, 0),
                 zc_s.at[s, 0], zc_r.at[s, 0], buddy).wait_recv()

        for j in range(3):
            src = srcs[j]
            s = slot(src)
            copy(sub(src, q_diag, 1), sub(src, q_diag, 1),
                 xr_s.at[s], xr_r.at[s], partner).wait_recv()
            copy(sub(src, q_diag, 0), sub(src, q_diag, 0),
                 zr_s.at[s], zr_r.at[s], buddy).wait_recv()

        for k in range(2):
            for t in range(3):
                y_t = (my_y + 1 + t) % N_Y
                copy(sub(my_y, q_me, k), sub(my_y, q_me, k),
                     ys_s.at[t, k], y_r.at[0, k],
                     (my_x, y_t, my_z)).wait_send()
        for j in range(3):
            src = srcs[j]
            s = slot(src)
            for k in range(2):
                copy(sub(src, q_me, k), sub(src, q_me, k),
                     xb_s.at[s, k], xb_r.at[s, k], partner).wait_send()
                copy(sub(src, q_me, k), sub(src, q_me, k),
                     zc_s.at[s, k], zc_r.at[s, k], buddy).wait_send()
            copy(sub(src, q_partner, 0), sub(src, q_partner, 0),
                 zr_s.at[s], zr_r.at[s], buddy).wait_send()
            copy(sub(src, q_buddy, 1), sub(src, q_buddy, 1),
                 xr_s.at[s], xr_r.at[s], partner).wait_send()

    dma = pltpu.SemaphoreType.DMA
    return pl.pallas_call(
        body,
        out_shape=jax.ShapeDtypeStruct((N_Y * m_per, n), jnp.bfloat16),
        in_specs=[pl.BlockSpec(memory_space=pltpu.VMEM)],
        out_specs=pl.BlockSpec(memory_space=pltpu.VMEM),
        scratch_shapes=[
            dma((3, 2)), dma((3, 2)),
            dma((3, 2)), dma((3, 2)),
            dma((3, 2)), dma((3, 2)),
            dma((3,)), dma((3,)),
            dma((3,)), dma((3,)),
        ],
        compiler_params=pltpu.CompilerParams(collective_id=0),
    )(x)


# device time: 51261 ns/iter; 1.0003x vs baseline; 1.0003x over previous
import jax
import jax.numpy as jnp
from jax import lax
from jax.experimental import pallas as pl
from jax.experimental.pallas import tpu as pltpu

N_Y = 4


def kernel(x):
    m_per, n = x.shape
    qrows = m_per // 4
    srows = qrows // 2

    def body(x_ref, out_ref, ys_s, y_r, xb_s, xb_r, zc_s, zc_r,
             xr_s, xr_r, zr_s, zr_r):
        my_x = lax.axis_index("x")
        my_y = lax.axis_index("y")
        my_z = lax.axis_index("z")
        zp = my_z % 2
        partner = (1 - my_x, my_y, my_z)
        buddy = (my_x, my_y, my_z - 2 * zp + 1)

        q_me = 2 * my_x + zp
        q_partner = 2 * (1 - my_x) + zp
        q_buddy = 2 * my_x + (1 - zp)
        q_diag = 2 * (1 - my_x) + (1 - zp)

        def sub(c, q, k):
            return out_ref.at[pl.ds(c * m_per + q * qrows + k * srows, srows), :]

        def copy(src, dst, ssem, rsem, dev):
            return pltpu.make_async_remote_copy(
                src_ref=src, dst_ref=dst, send_sem=ssem, recv_sem=rsem,
                device_id=dev, device_id_type=pl.DeviceIdType.MESH,
            )

        def slot(src_y):
            return jnp.where(src_y < my_y, src_y, src_y - 1)

        def sel(table):
            v = jnp.int32(table[3])
            for yy in (2, 1, 0):
                v = jnp.where(my_y == yy, table[yy], v)
            return v

        srcs = [sel(t) for t in ([1, 0, 1, 2], [2, 2, 3, 1], [3, 3, 0, 0])]

        barrier_sem = pltpu.get_barrier_semaphore()
        peers = [(my_x, (my_y + 1 + k) % N_Y, my_z) for k in range(3)]
        peers += [partner, buddy]
        for dev in peers:
            pl.semaphore_signal(
                barrier_sem, inc=1, device_id=dev,
                device_id_type=pl.DeviceIdType.MESH,
            )
        pl.semaphore_wait(barrier_sem, len(peers))

        out_ref[pl.ds(my_y * m_per + q_me * qrows, qrows), :] = (
            x_ref[pl.ds(q_me * qrows, qrows), :].astype(jnp.bfloat16)
        )

        for k in range(2):
            for t in range(3):
                y_t = (my_y + 1 + t) % N_Y
                r_slot = jnp.where(my_y < y_t, my_y, my_y - 1)
                copy(sub(my_y, q_me, k), sub(my_y, q_me, k),
                     ys_s.at[t, k], y_r.at[r_slot, k],
                     (my_x, y_t, my_z)).start()

        for d in range(1, 4):
            q = (q_me + d) % 4
            out_ref[pl.ds(my_y * m_per + q * qrows, qrows), :] = (
                x_ref[pl.ds(q * qrows, qrows), :].astype(jnp.bfloat16)
            )

        for j in range(3):
            src = srcs[j]
            s = slot(src)
            for k in range(2):
                copy(sub(src, q_me, k), sub(src, q_me, k),
                     ys_s.at[0, k], y_r.at[s, k],
                     (my_x, src, my_z)).wait_recv()
                copy(sub(src, q_me, k), sub(src, q_me, k),
                     xb_s.at[s, k], xb_r.at[s, k], partner).start()
                copy(sub(src, q_me, k), sub(src, q_me, k),
                     zc_s.at[s, k], zc_r.at[s, k], buddy).start()

        for j in range(3):
            src = srcs[j]
            s = slot(src)
            copy(sub(src, q_partner, 0), sub(src, q_partner, 0),
                 xb_s.at[s, 0], xb_r.at[s, 0], partner).wait_recv()
            copy(sub(src, q_partner, 0), sub(src, q_partner, 0),
                 zr_s.at[s], zr_r.at[s], buddy).start()
            copy(sub(src, q_buddy, 1), sub(src, q_buddy, 1),
                 zc_s.at[s, 1], zc_r.at[s, 1], buddy).wait_recv()
            copy(sub(src, q_buddy, 1), sub(src, q_buddy, 1),
                 xr_s.at[s], xr_r.at[s], partner).start()

        for j in range(3):
            src = srcs[j]
            s = slot(src)
            copy(sub(src, q_partner, 1), sub(src, q_partner, 1),
                 xb_s.at[s, 1], xb_r.at[s, 1], partner).wait_recv()
            copy(sub(src, q_buddy, 0), sub(src, q_buddy, 0),
                 zc_s.at[s, 0], zc_r.at[s, 0], buddy).wait_recv()
            copy(sub(src, q_diag, 1), sub(src, q_diag, 1),
                 xr_s.at[s], xr_r.at[s], partner).wait_recv()
            copy(sub(src, q_diag, 0), sub(src, q_diag, 0),
                 zr_s.at[s], zr_r.at[s], buddy).wait_recv()

        for k in range(2):
            for t in range(3):
                y_t = (my_y + 1 + t) % N_Y
                copy(sub(my_y, q_me, k), sub(my_y, q_me, k),
                     ys_s.at[t, k], y_r.at[0, k],
                     (my_x, y_t, my_z)).wait_send()
        for j in range(3):
            src = srcs[j]
            s = slot(src)
            for k in range(2):
                copy(sub(src, q_me, k), sub(src, q_me, k),
                     xb_s.at[s, k], xb_r.at[s, k], partner).wait_send()
                copy(sub(src, q_me, k), sub(src, q_me, k),
                     zc_s.at[s, k], zc_r.at[s, k], buddy).wait_send()
            copy(sub(src, q_partner, 0), sub(src, q_partner, 0),
                 zr_s.at[s], zr_r.at[s], buddy).wait_send()
            copy(sub(src, q_buddy, 1), sub(src, q_buddy, 1),
                 xr_s.at[s], xr_r.at[s], partner).wait_send()

    dma = pltpu.SemaphoreType.DMA
    return pl.pallas_call(
        body,
        out_shape=jax.ShapeDtypeStruct((N_Y * m_per, n), jnp.bfloat16),
        in_specs=[pl.BlockSpec(memory_space=pltpu.VMEM)],
        out_specs=pl.BlockSpec(memory_space=pltpu.VMEM),
        scratch_shapes=[
            dma((3, 2)), dma((3, 2)),
            dma((3, 2)), dma((3, 2)),
            dma((3, 2)), dma((3, 2)),
            dma((3,)), dma((3,)),
            dma((3,)), dma((3,)),
        ],
        compiler_params=pltpu.CompilerParams(collective_id=0),
    )(x)
